# baseline (speedup 1.0000x reference)
import functools

import jax
import jax.numpy as jnp
from jax import lax
from jax.experimental import pallas as pl
from jax.experimental.pallas import tpu as pltpu
from jax.experimental.pallas import tpu_sc as plsc

NUM_HEADS = 16
SEQ = 1025
TBL_WORDS = 3972 * NUM_HEADS
NW = 32
NB = 4
ROW_PAD = 1040

T_ROW0 = 3969
T_COL0 = 3970
T_CORNER = 3971

SLABS = ((0, 256), (256, 256), (512, 256), (768, 257))


def _make_slab_kernel(i0, ni):
    total_rows = ni * NUM_HEADS
    if total_rows % NW == 0:
        rows_per = total_rows // NW
    else:
        rows_per = -((total_rows // NW + 1) // -NB) * NB
    n_iters = rows_per // NB
    assert n_iters * NB == rows_per

    @functools.partial(
        pl.kernel,
        mesh=plsc.VectorSubcoreMesh(core_axis_name="c", subcore_axis_name="s"),
        out_type=jax.ShapeDtypeStruct((ni, NUM_HEADS, SEQ), jnp.float32),
        scratch_types=(
            [pltpu.VMEM((TBL_WORDS,), jnp.float32)]
            + [pltpu.VMEM((ROW_PAD,), jnp.float32)] * NB
            + [pltpu.SemaphoreType.DMA] * NB
        ),
        compiler_params=pltpu.CompilerParams(
            needs_layout_passes=False, use_tc_tiling_on_sc=False,
            disable_bounds_checks=True),
    )
    def slab_kernel(tbl_hbm, out_hbm, tbl_v, *bufs_sems):
        bufs, sems = bufs_sems[:NB], bufs_sems[NB:]
        wid = lax.axis_index("s") * 2 + lax.axis_index("c")
        pltpu.sync_copy(tbl_hbm, tbl_v)

        if total_rows % NW == 0:
            start = wid * rows_per
        else:
            start = (wid * (total_rows - rows_per)) // (NW - 1)
        lane = lax.iota(jnp.int32, 16)
        lane0 = lane == 0
        edge = jnp.where(lane0, 31, 0).astype(jnp.int32)

        def build_row(buf, il, h):
            i = i0 + il
            hb = h * 3972

            def interior():
                p = i - 1
                r0 = p >> 5
                c0 = p & 31
                s = hb + (r0 + 31) * 63 + (c0 + 31)
                base0 = jnp.full((16,), s + 1, jnp.int32) - lane
                o0 = jnp.where(lane0, hb + T_COL0, base0)
                buf[pl.ds(0, 16)] = plsc.load_gather(tbl_v, [o0])
                for j in range(1, 64):
                    if j % 2 == 1:
                        off = base0 - (63 * ((j - 1) // 2) + 16)
                    else:
                        off = base0 - 63 * (j // 2) + edge
                    buf[pl.ds(16 * j, 16)] = plsc.load_gather(tbl_v, [off])
                o64 = jnp.full((16,), s - 1984, jnp.int32)
                buf[pl.ds(1024, 16)] = plsc.load_gather(tbl_v, [o64])

            if i0 == 0:
                @pl.when(i == 0)
                def _():
                    vspec = plsc.load_gather(
                        tbl_v, [jnp.where(lane0, hb + T_CORNER, hb + T_ROW0)])
                    buf[pl.ds(0, 16)] = vspec
                    vfill = plsc.load_gather(
                        tbl_v, [jnp.full((16,), hb + T_ROW0, jnp.int32)])
                    for j in range(1, 65):
                        buf[pl.ds(16 * j, 16)] = vfill

                pl.when(i != 0)(interior)
            else:
                interior()

        def do_iter(it, carry):
            for b in range(NB):
                buf, sem = bufs[b], sems[b]

                @pl.when(it > 0)
                def _():
                    pltpu.make_async_copy(
                        buf.at[pl.ds(0, SEQ)], out_hbm.at[0, 0], sem).wait()

                r = start + NB * it + b
                il = r >> 4
                h = r & 15
                build_row(buf, il, h)
                pltpu.async_copy(
                    buf.at[pl.ds(0, SEQ)], out_hbm.at[il, h], sem)
            return carry

        lax.fori_loop(0, n_iters, do_iter, 0)
        for b in range(NB):
            pltpu.make_async_copy(
                bufs[b].at[pl.ds(0, SEQ)], out_hbm.at[0, 0], sems[b]).wait()

    return slab_kernel


_SLAB_KERNELS = [_make_slab_kernel(i0, ni) for i0, ni in SLABS]


def kernel(relative_position_bias_table, window_size):
    del window_size
    tbl_flat = relative_position_bias_table.T.reshape(-1)
    slabs = [k(tbl_flat) for k in _SLAB_KERNELS]
    out = jnp.concatenate(slabs, axis=0)
    return out.transpose(1, 0, 2)[None]


# speedup vs baseline: 1.3008x; 1.3008x over previous
import functools

import jax
import jax.numpy as jnp
from jax import lax
from jax.experimental import pallas as pl
from jax.experimental.pallas import tpu as pltpu
from jax.experimental.pallas import tpu_sc as plsc

NUM_HEADS = 16
SEQ = 1025
TBL_WORDS = 3972 * NUM_HEADS
NW = 32
NB = 4
ROW_PAD = 1040

T_ROW0 = 3969
T_COL0 = 3970
T_CORNER = 3971

SLABS = ((0, 256), (256, 256), (512, 256), (768, 257))


def _make_slab_kernel(i0, ni):
    total_rows = ni * NUM_HEADS
    if total_rows % NW == 0:
        rows_per = total_rows // NW
    else:
        rows_per = -((total_rows // NW + 1) // -NB) * NB
    n_iters = rows_per // NB
    assert n_iters * NB == rows_per

    @functools.partial(
        pl.kernel,
        mesh=plsc.VectorSubcoreMesh(core_axis_name="c", subcore_axis_name="s"),
        out_type=jax.ShapeDtypeStruct((ni, NUM_HEADS, SEQ), jnp.float32),
        scratch_types=(
            [pltpu.VMEM((TBL_WORDS,), jnp.float32)]
            + [pltpu.VMEM((ROW_PAD,), jnp.float32)] * NB
            + [pltpu.SemaphoreType.DMA] * NB
        ),
        compiler_params=pltpu.CompilerParams(
            needs_layout_passes=False, use_tc_tiling_on_sc=False,
            disable_bounds_checks=True),
    )
    def slab_kernel(tbl_hbm, out_hbm, tbl_v, *bufs_sems):
        bufs, sems = bufs_sems[:NB], bufs_sems[NB:]
        wid = lax.axis_index("s") * 2 + lax.axis_index("c")
        pltpu.sync_copy(tbl_hbm, tbl_v)

        if total_rows % NW == 0:
            start = wid * rows_per
        else:
            start = (wid * (total_rows - rows_per)) // (NW - 1)
        lane = lax.iota(jnp.int32, 16)
        lane0 = lane == 0
        edge = jnp.where(lane0, 31, 0).astype(jnp.int32)
        neg16 = jnp.full((16,), -16, jnp.int32)

        def build_row(buf, il, h):
            i = i0 + il
            hb = h * 3972

            def interior():
                p = i - 1
                r0 = p >> 5
                c0 = p & 31
                s = hb + (r0 + 31) * 63 + (c0 + 31)
                base0 = jnp.full((16,), s + 1, jnp.int32) - lane
                o0 = jnp.where(lane0, hb + T_COL0, base0)
                buf[pl.ds(0, 16)] = plsc.load_gather(tbl_v, [o0])

                @plsc.parallel_loop(1, 64, unroll=8)
                def _(j):
                    j = j.astype(jnp.int32)
                    off = (base0 - 63 * (j >> 1)) + jnp.where(
                        (j & 1) == 1, neg16, edge)
                    buf[pl.ds(pl.multiple_of(j * 16, 16), 16)] = (
                        plsc.load_gather(tbl_v, [off]))

                o64 = jnp.full((16,), s - 1984, jnp.int32)
                buf[pl.ds(1024, 16)] = plsc.load_gather(tbl_v, [o64])

            if i0 == 0:
                @pl.when(i == 0)
                def _():
                    vspec = plsc.load_gather(
                        tbl_v, [jnp.where(lane0, hb + T_CORNER, hb + T_ROW0)])
                    buf[pl.ds(0, 16)] = vspec
                    vfill = plsc.load_gather(
                        tbl_v, [jnp.full((16,), hb + T_ROW0, jnp.int32)])
                    for j in range(1, 65):
                        buf[pl.ds(16 * j, 16)] = vfill

                pl.when(i != 0)(interior)
            else:
                interior()

        def do_iter(it, carry):
            for b in range(NB):
                buf, sem = bufs[b], sems[b]

                @pl.when(it > 0)
                def _():
                    pltpu.make_async_copy(
                        buf.at[pl.ds(0, SEQ)], out_hbm.at[0, 0], sem).wait()

                r = start + NB * it + b
                il = r >> 4
                h = r & 15
                build_row(buf, il, h)
                pltpu.async_copy(
                    buf.at[pl.ds(0, SEQ)], out_hbm.at[il, h], sem)
            return carry

        lax.fori_loop(0, n_iters, do_iter, 0)
        for b in range(NB):
            pltpu.make_async_copy(
                bufs[b].at[pl.ds(0, SEQ)], out_hbm.at[0, 0], sems[b]).wait()

    return slab_kernel


_SLAB_KERNELS = [_make_slab_kernel(i0, ni) for i0, ni in SLABS]


def kernel(relative_position_bias_table, window_size):
    del window_size
    tbl_flat = relative_position_bias_table.T.reshape(-1)
    slabs = [k(tbl_flat) for k in _SLAB_KERNELS]
    out = jnp.concatenate(slabs, axis=0)
    return out.transpose(1, 0, 2)[None]
